# baseline (speedup 1.0000x reference)
import functools

import jax
import jax.numpy as jnp
from jax import lax
from jax.experimental import pallas as pl
from jax.experimental.pallas import tpu as pltpu
from jax.experimental.pallas import tpu_sc as plsc

_L = 16
_NC = 2
_NS = 16
_NW = _NC * _NS
_C = 16
_NBUF = 7
_P = 2


def _sc_lookup(pe, idx3):
    n_chunks = idx3.shape[1]
    b_per_w = n_chunks * _C
    B = _NW * b_per_w
    D = pe.shape[1]
    mesh = plsc.VectorSubcoreMesh(core_axis_name="c", subcore_axis_name="s")

    @functools.partial(
        pl.kernel,
        mesh=mesh,
        compiler_params=pltpu.CompilerParams(needs_layout_passes=False),
        out_type=jax.ShapeDtypeStruct((B, D), jnp.float32),
        scratch_types=[
            pltpu.VMEM((n_chunks, _C), jnp.int32),
        ]
        + [pltpu.VMEM((_C, D), jnp.float32)] * _NBUF
        + [pltpu.SemaphoreType.DMA] * (2 * _NBUF),
    )
    def k(pe_hbm, idx_hbm, out_hbm, idx_v, *bufs_and_sems):
        bufs = bufs_and_sems[:_NBUF]
        g_sems = bufs_and_sems[_NBUF:2 * _NBUF]
        w_sems = bufs_and_sems[2 * _NBUF:]
        wid = lax.axis_index("s") * _NC + lax.axis_index("c")
        base = wid * b_per_w
        pltpu.sync_copy(idx_hbm.at[wid], idx_v)

        def start_gather(c, b):
            cc = jnp.minimum(c, n_chunks - 1)
            pltpu.async_copy(pe_hbm.at[idx_v.at[cc]], bufs[b], g_sems[b])

        def wait_gather(b):
            pltpu.make_async_copy(
                pe_hbm.at[pl.ds(0, _C)], bufs[b], g_sems[b]).wait()

        def start_writeback(c, b):
            off = pl.multiple_of(base + c * _C, _C)
            pltpu.async_copy(bufs[b], out_hbm.at[pl.ds(off, _C)], w_sems[b])

        def wait_writeback(b):
            pltpu.make_async_copy(
                bufs[b], out_hbm.at[pl.ds(0, _C)], w_sems[b]).wait()

        def fix_padding(c, b):
            buf = bufs[b]
            m = idx_v[c, pl.ds(0, _L)]
            for g in range(1, _C // _L):
                m = jnp.minimum(m, idx_v[c, pl.ds(g * _L, _L)])
            smallest = jnp.min(m)

            @pl.when(smallest <= 0)
            def _():
                def row_body(r, _):
                    splat = plsc.load_gather(
                        idx_v,
                        [jnp.full((_L,), c, jnp.int32),
                         jnp.full((_L,), r, jnp.int32)])
                    s = jnp.minimum(splat, 1).astype(jnp.float32)

                    def col_body(j, __):
                        off = pl.multiple_of(j * _L, _L)
                        buf[r, pl.ds(off, _L)] = buf[r, pl.ds(off, _L)] * s
                        return 0

                    return lax.fori_loop(0, D // _L, col_body, 0)

                lax.fori_loop(0, _C, row_body, 0)

        def slot(c, b, maybe_first):
            bn = (b + _P) % _NBUF
            wait_gather(b)
            fix_padding(c, b)
            start_writeback(c, b)
            if maybe_first:
                @pl.when(c >= _NBUF - _P)
                def _():
                    wait_writeback(bn)
            else:
                wait_writeback(bn)
            start_gather(c + _P, bn)

        for j in range(_P):
            start_gather(j, j)

        n_rings = n_chunks // _NBUF

        def body_first(i, _):
            for k_ in range(_NBUF):
                slot(i * _NBUF + k_, k_, maybe_first=True)
            return 0

        def body_steady(i, _):
            for k_ in range(_NBUF):
                slot(i * _NBUF + k_, k_, maybe_first=False)
            return 0

        lax.fori_loop(0, 1, body_first, 0)
        lax.fori_loop(1, n_rings, body_steady, 0)
        for c in range(n_rings * _NBUF, n_chunks):
            slot(c, c % _NBUF, maybe_first=False)
        for j in range(_NBUF - _P):
            wait_writeback((n_chunks - (_NBUF - _P) + j) % _NBUF)
        for j in range(_P):
            wait_gather((n_chunks + j) % _NBUF)

    return k(pe, idx3)


def kernel(x, pe):
    B4, S = x.shape
    B = B4 * S
    b_per_w = B // _NW
    n_chunks = b_per_w // _C
    idx3 = x.reshape(_NW, n_chunks, _C)
    out = _sc_lookup(pe, idx3)
    return out.reshape(B4, S, pe.shape[1])


# speedup vs baseline: 1.0127x; 1.0127x over previous
import functools

import jax
import jax.numpy as jnp
from jax import lax
from jax.experimental import pallas as pl
from jax.experimental.pallas import tpu as pltpu
from jax.experimental.pallas import tpu_sc as plsc

_L = 16
_NC = 2
_NS = 16
_NW = _NC * _NS
_C = 16
_NBUF = 7
_P = 3


def _sc_lookup(pe, idx3):
    n_chunks = idx3.shape[1]
    b_per_w = n_chunks * _C
    B = _NW * b_per_w
    D = pe.shape[1]
    mesh = plsc.VectorSubcoreMesh(core_axis_name="c", subcore_axis_name="s")

    @functools.partial(
        pl.kernel,
        mesh=mesh,
        compiler_params=pltpu.CompilerParams(needs_layout_passes=False),
        out_type=jax.ShapeDtypeStruct((B, D), jnp.float32),
        scratch_types=[
            pltpu.VMEM((n_chunks, _C), jnp.int32),
        ]
        + [pltpu.VMEM((_C, D), jnp.float32)] * _NBUF
        + [pltpu.SemaphoreType.DMA] * (2 * _NBUF),
    )
    def k(pe_hbm, idx_hbm, out_hbm, idx_v, *bufs_and_sems):
        bufs = bufs_and_sems[:_NBUF]
        g_sems = bufs_and_sems[_NBUF:2 * _NBUF]
        w_sems = bufs_and_sems[2 * _NBUF:]
        wid = lax.axis_index("s") * _NC + lax.axis_index("c")
        base = wid * b_per_w
        pltpu.sync_copy(idx_hbm.at[wid], idx_v)

        def start_gather(c, b):
            cc = jnp.minimum(c, n_chunks - 1)
            pltpu.async_copy(pe_hbm.at[idx_v.at[cc]], bufs[b], g_sems[b])

        def wait_gather(b):
            pltpu.make_async_copy(
                pe_hbm.at[pl.ds(0, _C)], bufs[b], g_sems[b]).wait()

        def start_writeback(c, b):
            off = pl.multiple_of(base + c * _C, _C)
            pltpu.async_copy(bufs[b], out_hbm.at[pl.ds(off, _C)], w_sems[b])

        def wait_writeback(b):
            pltpu.make_async_copy(
                bufs[b], out_hbm.at[pl.ds(0, _C)], w_sems[b]).wait()

        def fix_padding(c, b):
            buf = bufs[b]
            m = idx_v[c, pl.ds(0, _L)]
            for g in range(1, _C // _L):
                m = jnp.minimum(m, idx_v[c, pl.ds(g * _L, _L)])
            smallest = jnp.min(m)

            @pl.when(smallest <= 0)
            def _():
                def row_body(r, _):
                    splat = plsc.load_gather(
                        idx_v,
                        [jnp.full((_L,), c, jnp.int32),
                         jnp.full((_L,), r, jnp.int32)])
                    s = jnp.minimum(splat, 1).astype(jnp.float32)

                    def col_body(j, __):
                        off = pl.multiple_of(j * _L, _L)
                        buf[r, pl.ds(off, _L)] = buf[r, pl.ds(off, _L)] * s
                        return 0

                    return lax.fori_loop(0, D // _L, col_body, 0)

                lax.fori_loop(0, _C, row_body, 0)

        def slot(c, b, maybe_first):
            bn = (b + _P) % _NBUF
            wait_gather(b)
            fix_padding(c, b)
            start_writeback(c, b)
            if maybe_first:
                @pl.when(c >= _NBUF - _P)
                def _():
                    wait_writeback(bn)
            else:
                wait_writeback(bn)
            start_gather(c + _P, bn)

        for j in range(_P):
            start_gather(j, j)

        n_rings = n_chunks // _NBUF

        def body_first(i, _):
            for k_ in range(_NBUF):
                slot(i * _NBUF + k_, k_, maybe_first=True)
            return 0

        def body_steady(i, _):
            for k_ in range(_NBUF):
                slot(i * _NBUF + k_, k_, maybe_first=False)
            return 0

        lax.fori_loop(0, 1, body_first, 0)
        lax.fori_loop(1, n_rings, body_steady, 0)
        for c in range(n_rings * _NBUF, n_chunks):
            slot(c, c % _NBUF, maybe_first=False)
        for j in range(_NBUF - _P):
            wait_writeback((n_chunks - (_NBUF - _P) + j) % _NBUF)
        for j in range(_P):
            wait_gather((n_chunks + j) % _NBUF)

    return k(pe, idx3)


def kernel(x, pe):
    B4, S = x.shape
    B = B4 * S
    b_per_w = B // _NW
    n_chunks = b_per_w // _C
    idx3 = x.reshape(_NW, n_chunks, _C)
    out = _sc_lookup(pe, idx3)
    return out.reshape(B4, S, pe.shape[1])
